# baseline (speedup 1.0000x reference)
import functools

import jax
import jax.numpy as jnp
from jax import lax
from jax.experimental import pallas as pl
from jax.experimental.pallas import tpu as pltpu
from jax.experimental.pallas import tpu_sc as plsc

B = 16384
D_MF = 8
D_MLP = 32
NC = 2
NS = 16
NW = NC * NS
BPW = B // NW

_sc_mesh = plsc.VectorSubcoreMesh(core_axis_name="c", subcore_axis_name="s")


@functools.partial(
    pl.kernel,
    mesh=_sc_mesh,
    compiler_params=pltpu.CompilerParams(use_tc_tiling_on_sc=False),
    out_type=[
        jax.ShapeDtypeStruct((B, D_MF), jnp.float32),
        jax.ShapeDtypeStruct((B, D_MF), jnp.float32),
        jax.ShapeDtypeStruct((B, D_MLP), jnp.float32),
        jax.ShapeDtypeStruct((B, D_MLP), jnp.float32),
    ],
    scratch_types=[
        pltpu.VMEM((BPW,), jnp.int32),
        pltpu.VMEM((BPW,), jnp.int32),
        pltpu.VMEM((BPW, D_MF), jnp.float32),
        pltpu.VMEM((BPW, D_MF), jnp.float32),
        pltpu.VMEM((BPW, D_MLP), jnp.float32),
        pltpu.VMEM((BPW, D_MLP), jnp.float32),
        pltpu.SemaphoreType.DMA,
        pltpu.SemaphoreType.DMA,
        pltpu.SemaphoreType.DMA,
        pltpu.SemaphoreType.DMA,
    ],
)
def _sc_gather(uid_hbm, iid_hbm, umf_hbm, imf_hbm, umlp_hbm, imlp_hbm,
               ue_mf_out, ie_mf_out, ue_mlp_out, ie_mlp_out,
               uidx_v, iidx_v, umf_v, imf_v, umlp_v, imlp_v,
               s0, s1, s2, s3):
    wid = lax.axis_index("s") * NC + lax.axis_index("c")
    base = wid * BPW
    pltpu.sync_copy(uid_hbm.at[pl.ds(base, BPW)], uidx_v)
    pltpu.sync_copy(iid_hbm.at[pl.ds(base, BPW)], iidx_v)
    c0 = pltpu.async_copy(umf_hbm.at[uidx_v], umf_v, s0)
    c1 = pltpu.async_copy(imf_hbm.at[iidx_v], imf_v, s1)
    c2 = pltpu.async_copy(umlp_hbm.at[uidx_v], umlp_v, s2)
    c3 = pltpu.async_copy(imlp_hbm.at[iidx_v], imlp_v, s3)
    c0.wait()
    pltpu.sync_copy(umf_v, ue_mf_out.at[pl.ds(base, BPW)])
    c1.wait()
    pltpu.sync_copy(imf_v, ie_mf_out.at[pl.ds(base, BPW)])
    c2.wait()
    pltpu.sync_copy(umlp_v, ue_mlp_out.at[pl.ds(base, BPW)])
    c3.wait()
    pltpu.sync_copy(imlp_v, ie_mlp_out.at[pl.ds(base, BPW)])


def _tc_mlp_body(ue_mf, ie_mf, ue_mlp, ie_mlp,
                 w1a, w1b, b1, w2, b2, wo_mf, wo_h, bo, out):
    h = jnp.dot(ue_mlp[...], w1a[...], preferred_element_type=jnp.float32)
    h = h + jnp.dot(ie_mlp[...], w1b[...], preferred_element_type=jnp.float32)
    h = jnp.maximum(h + b1[...], 0.0)
    h = jnp.dot(h, w2[...], preferred_element_type=jnp.float32) + b2[...]
    h = jnp.maximum(h, 0.0)
    mf = ue_mf[...] * ie_mf[...]
    o = jnp.dot(mf, wo_mf[...], preferred_element_type=jnp.float32)
    o = o + jnp.dot(h, wo_h[...], preferred_element_type=jnp.float32)
    out[...] = o + bo[...]


def _tc_mlp(ue_mf, ie_mf, ue_mlp, ie_mlp, W1, b1, W2, b2, Wo, bo):
    w1a = W1[:D_MLP]
    w1b = W1[D_MLP:]
    wo_mf = Wo[:D_MF]
    wo_h = Wo[D_MF:]
    return pl.pallas_call(
        _tc_mlp_body,
        out_shape=jax.ShapeDtypeStruct((B, 1), jnp.float32),
    )(ue_mf, ie_mf, ue_mlp, ie_mlp,
      w1a, w1b, b1.reshape(1, -1), W2, b2.reshape(1, -1),
      wo_mf, wo_h, bo.reshape(1, 1))


def kernel(user_ids, item_ids, user_mf, item_mf, user_mlp, item_mlp,
           W1, b1, W2, b2, Wo, bo):
    uid = user_ids.astype(jnp.int32)
    iid = item_ids.astype(jnp.int32)
    ue_mf, ie_mf, ue_mlp, ie_mlp = _sc_gather(
        uid, iid, user_mf, item_mf, user_mlp, item_mlp)
    out = _tc_mlp(ue_mf, ie_mf, ue_mlp, ie_mlp, W1, b1, W2, b2, Wo, bo)
    return out[:, 0]
